# baseline (speedup 1.0000x reference)
import functools

import jax
import jax.numpy as jnp
from jax import lax
from jax.experimental import pallas as pl
from jax.experimental.pallas import tpu as pltpu
from jax.experimental.pallas import tpu_sc as plsc

_RATIO = 0.7
_LMBDA = 0.5
_B = 16384
_C = 1000
_K = int(_B * _RATIO)

_BSC = 4096
_BT = _B - _BSC
_BN = 2048
_NB = _BT // _BN

_NW = 32
_CPW = _BSC // _NW
_TW = 16
_NT = _CPW // _TW


def _tc_phase1_body(yh_ref, y_ref, nll_ref, tl_ref):
    yh = yh_ref[...]
    yy = y_ref[...]
    m = jnp.max(yy, axis=0, keepdims=True)
    ii = lax.broadcasted_iota(jnp.int32, yy.shape, 0)
    idx = jnp.min(jnp.where(yy == m, ii, _C), axis=0, keepdims=True)
    nll = -jnp.sum(jnp.where(ii == idx, yh, 0.0), axis=0, keepdims=True)
    dot = jnp.sum(yh * yy, axis=0, keepdims=True)
    nll_ref[...] = nll
    tl_ref[...] = nll + _LMBDA * (1.0 - dot)


def _sc_phase1_body(yh_hbm, y_hbm, nll_hbm, tl_hbm,
                    byh0, byy0, byh1, byy1, snll, stl,
                    semh0, semy0, semh1, semy1):
    wid = lax.axis_index("s") * 2 + lax.axis_index("c")
    col_base = _BT + wid * _CPW
    bufs = ((byh0, byy0, semh0, semy0), (byh1, byy1, semh1, semy1))

    def copies(t, b):
        tile = (col_base // _TW) + t
        byh, byy, semh, semy = bufs[b]
        return (pltpu.make_async_copy(yh_hbm.at[:, tile, :], byh, semh),
                pltpu.make_async_copy(y_hbm.at[:, tile, :], byy, semy))

    def start(t, b):
        ch, cy = copies(t, b)
        ch.start()
        cy.start()

    def wait(t, b):
        ch, cy = copies(t, b)
        ch.wait()
        cy.wait()

    def compute(t, b):
        byh, byy, _, _ = bufs[b]

        def class_body(j, carry):
            dot, ym, yhm = carry
            yy = byy[j, :]
            yh = byh[j, :]
            upd = yy > ym
            ym = jnp.where(upd, yy, ym)
            yhm = jnp.where(upd, yh, yhm)
            dot = dot + yh * yy
            return dot, ym, yhm

        init = (jnp.zeros((16,), jnp.float32),
                jnp.full((16,), -1.0, jnp.float32),
                jnp.zeros((16,), jnp.float32))
        dot, _, yhm = lax.fori_loop(0, _C, class_body, init)
        nll = -yhm
        snll[pl.ds(t * _TW, _TW)] = nll
        stl[pl.ds(t * _TW, _TW)] = nll + _LMBDA * (1.0 - dot)

    start(0, 0)

    def outer(t2, _):
        for b in (0, 1):
            t = 2 * t2 + b

            @pl.when(t + 1 < _NT)
            def _():
                start(t + 1, 1 - b)

            wait(t, b)
            compute(t, b)
        return 0

    lax.fori_loop(0, _NT // 2, outer, 0)
    pltpu.sync_copy(snll, nll_hbm.at[pl.ds(wid * _CPW, _CPW)])
    pltpu.sync_copy(stl, tl_hbm.at[pl.ds(wid * _CPW, _CPW)])


_sc_phase1 = functools.partial(
    pl.kernel,
    out_type=[
        jax.ShapeDtypeStruct((_BSC,), jnp.float32),
        jax.ShapeDtypeStruct((_BSC,), jnp.float32),
    ],
    mesh=plsc.VectorSubcoreMesh(core_axis_name="c", subcore_axis_name="s"),
    scratch_types=[
        pltpu.VMEM((_C, _TW), jnp.float32),
        pltpu.VMEM((_C, _TW), jnp.float32),
        pltpu.VMEM((_C, _TW), jnp.float32),
        pltpu.VMEM((_C, _TW), jnp.float32),
        pltpu.VMEM((_CPW,), jnp.float32),
        pltpu.VMEM((_CPW,), jnp.float32),
        pltpu.SemaphoreType.DMA,
        pltpu.SemaphoreType.DMA,
        pltpu.SemaphoreType.DMA,
        pltpu.SemaphoreType.DMA,
    ],
    compiler_params=pltpu.CompilerParams(needs_layout_passes=False,
                                         use_tc_tiling_on_sc=False),
)(_sc_phase1_body)


def _phase2_body(nll_ref, tl_ref, out_ref):
    nll = nll_ref[...]
    tl = tl_ref[...]
    i32 = lax.bitcast_convert_type(tl, jnp.int32)
    keyi = jnp.where(i32 < 0, jnp.bitwise_not(i32),
                     jnp.bitwise_or(i32, jnp.int32(-(2**31))))
    u = lax.bitcast_convert_type(keyi, jnp.uint32)
    t = jnp.uint32(0)
    for b in range(31, -1, -1):
        cand = t | jnp.uint32(1 << b)
        cnt = jnp.sum((u >= cand).astype(jnp.int32))
        t = jnp.where(cnt >= _K, cand, t)
    gt = u > t
    eq = u == t
    cnt_gt = jnp.sum(gt.astype(jnp.int32))
    sum_gt = jnp.sum(jnp.where(gt, nll, 0.0))
    cnt_eq = jnp.sum(eq.astype(jnp.int32))
    sum_eq = jnp.sum(jnp.where(eq, nll, 0.0))
    rem = (_K - cnt_gt).astype(jnp.float32)
    total = sum_gt + rem * sum_eq / jnp.maximum(cnt_eq, 1).astype(jnp.float32)
    out_ref[...] = jnp.broadcast_to(total / jnp.float32(_K), (1, 1))


def kernel(y_hat, y):
    yht = y_hat.T
    yt = y.T
    nll_sc, tl_sc = _sc_phase1(yht.reshape(_C, _B // _TW, _TW),
                               yt.reshape(_C, _B // _TW, _TW))
    nll_tc, tl_tc = pl.pallas_call(
        _tc_phase1_body,
        grid=(_NB,),
        in_specs=[
            pl.BlockSpec((_C, _BN), lambda i: (0, i)),
            pl.BlockSpec((_C, _BN), lambda i: (0, i)),
        ],
        out_specs=[
            pl.BlockSpec((1, _BN), lambda i: (0, i)),
            pl.BlockSpec((1, _BN), lambda i: (0, i)),
        ],
        out_shape=[
            jax.ShapeDtypeStruct((1, _BT), jnp.float32),
            jax.ShapeDtypeStruct((1, _BT), jnp.float32),
        ],
    )(yht, yt)

    nll2 = jnp.concatenate([nll_tc.reshape(-1), nll_sc]).reshape(128, 128)
    tl2 = jnp.concatenate([tl_tc.reshape(-1), tl_sc]).reshape(128, 128)
    out = pl.pallas_call(
        _phase2_body,
        out_shape=jax.ShapeDtypeStruct((1, 1), jnp.float32),
    )(nll2, tl2)
    return out[0, 0]
